# Initial kernel scaffold; baseline (speedup 1.0000x reference)
import jax
import jax.numpy as jnp
from jax.experimental import pallas as pl


def kernel(pc1, pc2, params):
    raise NotImplementedError("write your pallas kernel here")


# speedup vs baseline: 1.0000x; 1.0000x over previous
import jax
import jax.numpy as jnp
import numpy as np
from jax.experimental import pallas as pl

N_ITERS = 4


def _lrelu(x):
    return jax.nn.leaky_relu(x, 0.1)


def _knn(input_xyz, query_xyz, k):
    d = (jnp.sum(query_xyz ** 2, axis=1)[:, :, None]
         + jnp.sum(input_xyz ** 2, axis=1)[:, None, :]
         - 2.0 * jnp.einsum('bcq,bci->bqi', query_xyz, input_xyz))
    _, idx = jax.lax.top_k(-d, k)
    return idx


def _gather_cf(data, idx):
    B, C = data.shape[0], data.shape[1]
    m = int(np.prod(idx.shape[1:]))
    flat = jnp.broadcast_to(idx.reshape(B, 1, m), (B, C, m))
    out = jnp.take_along_axis(data, flat, axis=2)
    return out.reshape((B, C) + idx.shape[1:])


def _conv1d(p, x):
    return jnp.einsum('oc,bcn->bon', p['w'], x) + p['b'][None, :, None]


def _conv2d(p, x):
    return jnp.einsum('oc,bcnk->bonk', p['w'], x) + p['b'][None, :, None, None]


def _mlp1d(ps, x):
    for p in ps:
        x = _lrelu(_conv1d(p, x))
    return x


def _mlp2d_relu(ps, x):
    for p in ps:
        x = jax.nn.relu(_conv2d(p, x))
    return x


def _point_conv(p, xyz_in, feat_in, xyz_q, k=16):
    idx = _knn(xyz_in, xyz_q, k)
    rel = _gather_cf(xyz_in, idx) - xyz_q[:, :, :, None]
    nfeat = _gather_cf(feat_in, idx)
    g = jnp.concatenate([rel, nfeat], axis=1)
    g = _conv2d(p, g)
    g = g * p['gamma'][None, :, None, None] + p['beta'][None, :, None, None]
    g = _lrelu(g)
    return jnp.max(g, axis=-1)


def _point_conv_dw(p, xyz, feat, k, knn_indices=None, act=True):
    if knn_indices is None:
        idx = _knn(xyz, xyz, k)
    else:
        idx = knn_indices[:, :, :k]
    lin = _conv1d(p['lin'], feat)
    nfeat = _gather_cf(lin, idx)
    rel = _gather_cf(xyz, idx) - xyz[:, :, :, None]
    w = _conv2d(p['wnet'], rel)
    out = jnp.mean(w * nfeat, axis=-1)
    if act:
        out = _lrelu(out)
    return out


def _encoder3d(p, xyzs):
    feats = [_mlp1d(p['level0'], xyzs[0])]
    for i in range(len(xyzs) - 1):
        f = _mlp1d(p['mlps'][i], feats[-1])
        f = _point_conv(p['convs'][i], xyzs[i], f, xyzs[i + 1], k=16)
        feats.append(f)
    return feats


def _knn_interpolation(input_xyz, input_feat, query_xyz, k=3):
    idx = _knn(input_xyz, query_xyz, k)
    nxyz = _gather_cf(input_xyz, idx)
    d = jnp.sqrt(jnp.sum((nxyz - query_xyz[:, :, :, None]) ** 2, axis=1) + 1e-12)
    w = 1.0 / (d + 1e-8)
    w = w / jnp.sum(w, axis=-1, keepdims=True)
    nfeat = _gather_cf(input_feat, idx)
    return jnp.sum(nfeat * w[:, None, :, :], axis=-1)


def _backwarp_3d(xyz1, xyz2, flow1, k=3):
    flow2 = _knn_interpolation(xyz1, flow1, xyz2, k=k)
    return xyz2 - flow2


def _build_cost_volume_pyramid(feat1, feat2, xyzs2, k=3):
    cv0 = jnp.einsum('bcn,bcm->bnm', feat1, feat2) / feat1.shape[1]
    cvs = [cv0]
    for i in range(1, len(xyzs2)):
        idx = _knn(xyzs2[i - 1], xyzs2[i], k)
        knn_corr = _gather_cf(cvs[i - 1], idx)
        cvs.append(jnp.mean(knn_corr, axis=-1))
    return cvs


def _calc_matching_cost(p, xyz1, xyz2, cv, k=16):
    idx = _knn(xyz2, xyz1, k)
    knn_xyz2 = _gather_cf(xyz2, idx)
    knn_xyz2_norm = knn_xyz2 - xyz1[:, :, :, None]
    knn_corr = jnp.take_along_axis(cv, idx, axis=2)[:, None, :, :]
    cost = _mlp2d_relu(p['cost_mlp'], jnp.concatenate([knn_xyz2_norm, knn_corr], axis=1))
    return jnp.sum(cost, axis=-1)


def _correlation3d(p, xyz1, xyzs2, cvs):
    costs = [_calc_matching_cost(p, xyz1, xyzs2[i], cvs[i], k=16) for i in range(4)]
    costs = jnp.concatenate(costs, axis=1)
    return _lrelu(_conv1d(p['merge'], costs))


def _motion_encoder(p, xyz, flow, corr, knn_idx):
    corr_feat = _point_conv_dw(p['conv_c1'], xyz, corr, 16, knn_idx)
    flow_feat = _point_conv_dw(p['conv_f1'], xyz, flow, 32, knn_idx)
    flow_feat = _point_conv_dw(p['conv_f2'], xyz, flow_feat, 16, knn_idx)
    out = _point_conv_dw(p['conv'], xyz, jnp.concatenate([corr_feat, flow_feat], axis=1), 16, knn_idx)
    return jnp.concatenate([out, flow], axis=1)


def _gru3d(p, xyz, h, x, knn_idx):
    hx = jnp.concatenate([h, x], axis=1)
    z = jax.nn.sigmoid(_point_conv_dw(p['conv_z'], xyz, hx, 4, knn_idx, act=False))
    r = jax.nn.sigmoid(_point_conv_dw(p['conv_r'], xyz, hx, 4, knn_idx, act=False))
    q = jnp.tanh(_point_conv_dw(p['conv_q'], xyz, jnp.concatenate([r * h, x], axis=1), 4, knn_idx, act=False))
    return (1 - z) * h + z * q


def _flow_head(p, xyz, feat, knn_idx):
    f = _point_conv_dw(p['conv1'], xyz, feat, 32, knn_idx)
    f = _point_conv_dw(p['conv2'], xyz, f, 32, knn_idx)
    return _conv1d(p['fc'], f)


def _build_pc_pyramid(pc, n_samples):
    xyzs = [pc]
    cur = pc
    for n in n_samples:
        cur = cur[:, :, :n]
        xyzs.append(cur)
    return xyzs


def _identity_kernel(x_ref, o_ref):
    o_ref[...] = x_ref[...]


def _pallas_identity(x):
    return pl.pallas_call(
        _identity_kernel,
        out_shape=jax.ShapeDtypeStruct(x.shape, x.dtype),
    )(x)


def kernel(pc1, pc2, params):
    xyzs1 = _build_pc_pyramid(pc1, [4096, 2048, 1024, 512, 256])
    xyzs2 = _build_pc_pyramid(pc2, [4096, 2048, 1024, 512, 256])
    feat1 = _encoder3d(params['fnet'], xyzs1[:3])[2]
    feat2 = _encoder3d(params['fnet'], xyzs2[:3])[2]
    featc = _encoder3d(params['cnet'], xyzs1[:3])[2]
    featc = _conv1d(params['cnet_aligner'], featc)
    xyzs1, xyzs2 = xyzs1[2:], xyzs2[2:]
    xyz1 = xyzs1[0]
    cvs = _build_cost_volume_pyramid(feat1, feat2, xyzs2, k=3)
    h = jnp.tanh(featc[:, :128, :])
    x = jax.nn.relu(featc[:, 128:, :])
    knn_idx = _knn(xyz1, xyz1, 32)
    flow_preds = []
    flow_pred = jnp.zeros_like(xyz1)
    for it in range(N_ITERS):
        if it > 0:
            flow_pred = jax.lax.stop_gradient(flow_pred)
            xyzs2_warp = [_backwarp_3d(xyz1, l, flow_pred) for l in xyzs2]
        else:
            flow_pred = jnp.zeros_like(xyz1)
            xyzs2_warp = xyzs2
        corr = _correlation3d(params['correlation'], xyz1, xyzs2_warp, cvs)
        motion_feat = _motion_encoder(params['motion_encoder'], xyz1, flow_pred, corr, knn_idx)
        h = _gru3d(params['gru'], xyz1, h, jnp.concatenate([x, motion_feat], axis=1), knn_idx)
        flow_delta = _flow_head(params['flow_head'], xyz1, h, knn_idx)
        flow_pred = flow_pred + flow_delta
        flow_preds.append(flow_pred)
    ups = [_knn_interpolation(xyz1, fp, pc1, k=3) for fp in flow_preds]
    out = jnp.stack(ups, axis=0)
    return _pallas_identity(out)


# speedup vs baseline: 52.4255x; 52.4253x over previous
import functools
import jax
import jax.numpy as jnp
import numpy as np
from jax import lax
from jax.experimental import pallas as pl
from jax.experimental.pallas import tpu as pltpu
from jax.experimental.pallas import tpu_sc as plsc

N_ITERS = 4
_NW = 32


def _sc_gather_multi(specs):
    n = len(specs)
    budget = 98304 // n
    plans = []
    for table, idx in specs:
        V, D = table.shape
        (M,) = idx.shape
        assert D % 128 == 0 and M % (8 * _NW) == 0, (V, D, M)
        b_per_w = M // _NW
        ch = b_per_w
        while ch * D > budget or ch > 2048:
            ch //= 2
        if ch % 8 != 0 or b_per_w % ch != 0:
            ch = 8
        assert ch >= 8 and b_per_w % ch == 0, (b_per_w, ch, D)
        plans.append((V, D, M, b_per_w, ch))

    scratch = []
    for (V, D, M, b_per_w, ch) in plans:
        scratch.append(pltpu.VMEM((ch,), jnp.int32))
        scratch.append(pltpu.VMEM((ch, D), jnp.float32))
    scratch.append(pltpu.SemaphoreType.DMA)

    mesh = plsc.VectorSubcoreMesh(core_axis_name="c", subcore_axis_name="s")

    @functools.partial(
        pl.kernel, mesh=mesh,
        out_type=[jax.ShapeDtypeStruct((M, D), jnp.float32)
                  for (V, D, M, b, c) in plans],
        scratch_types=scratch,
    )
    def k(*refs):
        ins = refs[:2 * n]
        outs = refs[2 * n:3 * n]
        scr = refs[3 * n:]
        sem = scr[-1]
        wid = lax.axis_index("s") * 2 + lax.axis_index("c")
        for t in range(n):
            V, D, M, b_per_w, ch = plans[t]
            table_hbm = ins[2 * t]
            idx_hbm = ins[2 * t + 1]
            out_hbm = outs[t]
            idx_v = scr[2 * t]
            rows_v = scr[2 * t + 1]
            base = wid * b_per_w

            def body(i, _, table_hbm=table_hbm, idx_hbm=idx_hbm,
                     out_hbm=out_hbm, idx_v=idx_v, rows_v=rows_v,
                     base=base, ch=ch):
                off = base + i * ch
                pltpu.sync_copy(idx_hbm.at[pl.ds(off, ch)], idx_v)
                pltpu.async_copy(table_hbm.at[idx_v], rows_v, sem).wait()
                pltpu.sync_copy(rows_v, out_hbm.at[pl.ds(off, ch)])
                return 0

            lax.fori_loop(0, b_per_w // ch, body, 0, unroll=False)

    flat_in = []
    for table, idx in specs:
        flat_in += [table, idx]
    outs = k(*flat_in)
    return list(outs) if isinstance(outs, (list, tuple)) else [outs]


def _flat_idx(idx, V):
    B, Nq, k = idx.shape
    off = (jnp.arange(B, dtype=jnp.int32) * V)[:, None, None]
    return jnp.transpose(idx + off, (2, 0, 1)).reshape(-1)


def _stack_rows(x):
    B, C, N = x.shape
    return jnp.transpose(x, (0, 2, 1)).reshape(B * N, C)


def _pad128(t):
    D = t.shape[-1]
    Dp = ((D + 127) // 128) * 128
    if Dp == D:
        return t
    return jnp.pad(t, ((0, 0), (0, Dp - D)))


def _lrelu(x):
    return jax.nn.leaky_relu(x, 0.1)


def _knn(input_xyz, query_xyz, k):
    d = (jnp.sum(query_xyz ** 2, axis=1)[:, :, None]
         + jnp.sum(input_xyz ** 2, axis=1)[:, None, :]
         - 2.0 * jnp.einsum('bcq,bci->bqi', query_xyz, input_xyz))
    _, idx = jax.lax.top_k(-d, k)
    return idx.astype(jnp.int32)


def _conv1d(p, x):
    return jnp.einsum('oc,bcn->bon', p['w'], x) + p['b'][None, :, None]


def _mlp1d(ps, x):
    for p in ps:
        x = _lrelu(_conv1d(p, x))
    return x


def _build_pc_pyramid(pc, n_samples):
    xyzs = [pc]
    cur = pc
    for n in n_samples:
        cur = cur[:, :, :n]
        xyzs.append(cur)
    return xyzs


def _point_conv_post(p, g, xyz_q, C):
    rel = g[..., :3] - jnp.transpose(xyz_q, (0, 2, 1))[None]
    nfeat = g[..., 3:3 + C]
    cat = jnp.concatenate([rel, nfeat], axis=-1)
    h = jnp.einsum('jbnc,oc->jbno', cat, p['w']) + p['b']
    h = h * p['gamma'] + p['beta']
    h = _lrelu(h)
    h = jnp.max(h, axis=0)
    return jnp.transpose(h, (0, 2, 1))


def _dw_post(p, nf, rel, act=True):
    C = p['wnet']['w'].shape[0]
    w = jnp.einsum('jbnd,od->jbno', rel, p['wnet']['w']) + p['wnet']['b']
    out = jnp.mean(w * nf[..., :C], axis=0)
    out = jnp.transpose(out, (0, 2, 1))
    if act:
        out = _lrelu(out)
    return out


def _interp_post(g, query_xyz):
    nxyz = g[..., :3]
    nfeat = g[..., 3:6]
    q = jnp.transpose(query_xyz, (0, 2, 1))[None]
    d = jnp.sqrt(jnp.sum((nxyz - q) ** 2, axis=-1) + 1e-12)
    w = 1.0 / (d + 1e-8)
    w = w / jnp.sum(w, axis=0, keepdims=True)
    out = jnp.sum(nfeat * w[..., None], axis=0)
    return jnp.transpose(out, (0, 2, 1))


def kernel(pc1, pc2, params):
    B = pc1.shape[0]
    xyzs1 = _build_pc_pyramid(pc1, [4096, 2048, 1024, 512, 256])
    xyzs2 = _build_pc_pyramid(pc2, [4096, 2048, 1024, 512, 256])

    def enc_level0(p, xyz0):
        return _mlp1d(p['level0'], xyz0)

    encs = [(params['fnet'], xyzs1), (params['fnet'], xyzs2),
            (params['cnet'], xyzs1)]
    f_lvl = [_mlp1d(e[0]['mlps'][0], enc_level0(e[0], e[1][0])) for e in encs]

    specs = []
    for (p, xz), f in zip(encs, f_lvl):
        idx = _knn(xz[0], xz[1], 16)
        table = _pad128(jnp.concatenate(
            [_stack_rows(xz[0]), _stack_rows(f)], axis=-1))
        specs.append((table, _flat_idx(idx, xz[0].shape[2])))
    gs = _sc_gather_multi(specs)

    feats1 = []
    for (p, xz), g in zip(encs, gs):
        Nq = xz[1].shape[2]
        gg = g.reshape(16, B, Nq, -1)
        feats1.append(_point_conv_post(p['convs'][0], gg, xz[1], 96))

    f_lvl2 = [_mlp1d(e[0]['mlps'][1], f) for e, f in zip(encs, feats1)]

    xyz1 = xyzs1[2]
    specs = []
    for (p, xz), f in zip(encs, f_lvl2):
        idx = _knn(xz[1], xz[2], 16)
        table = _pad128(jnp.concatenate(
            [_stack_rows(xz[1]), _stack_rows(f)], axis=-1))
        specs.append((table, _flat_idx(idx, xz[1].shape[2])))
    knn_idx = _knn(xyz1, xyz1, 32)
    xyz1_rows = _pad128(_stack_rows(xyz1))
    specs.append((xyz1_rows, _flat_idx(knn_idx, 2048)))
    gs = _sc_gather_multi(specs)

    feats2 = []
    for (p, xz), g in zip(encs, gs[:3]):
        gg = g.reshape(16, B, 2048, -1)
        feats2.append(_point_conv_post(p['convs'][1], gg, xz[2], 128))
    feat1, feat2, featc = feats2
    featc = _conv1d(params['cnet_aligner'], featc)

    rel32 = gs[3].reshape(32, B, 2048, -1)[..., :3] \
        - jnp.transpose(xyz1, (0, 2, 1))[None]

    cvT = [jnp.einsum('bcm,bcn->bmn', feat2, feat1) / 128.0]
    Ns2 = [2048, 1024, 512, 256]
    for i in range(1, 4):
        idx = _knn(xyzs2[2 + i - 1], xyzs2[2 + i], 3)
        table = cvT[i - 1].reshape(B * Ns2[i - 1], 2048)
        g = _sc_gather_multi([(table, _flat_idx(idx, Ns2[i - 1]))])[0]
        cvT.append(jnp.mean(g.reshape(3, B, Ns2[i], 2048), axis=0))
    cv_flat = [c.reshape(B * Ns2[i] * 16, 128) for i, c in enumerate(cvT)]

    h = jnp.tanh(featc[:, :128, :])
    x = jax.nn.relu(featc[:, 128:, :])

    xyzs2c = xyzs2[2:]
    lane_eye = jnp.eye(128, dtype=jnp.float32)
    n_idx = jnp.arange(2048, dtype=jnp.int32)

    mp = params['motion_encoder']
    gp = params['gru']
    fp = params['flow_head']
    cp = params['correlation']

    flow_preds = []
    flow_pred = jnp.zeros_like(xyz1)
    for it in range(N_ITERS):
        if it > 0:
            table = _pad128(jnp.concatenate(
                [_stack_rows(xyz1), _stack_rows(flow_pred)], axis=-1))
            specs = []
            for l in xyzs2c:
                idx = _knn(xyz1, l, 3)
                specs.append((table, _flat_idx(idx, 2048)))
            gs = _sc_gather_multi(specs)
            xyzs2_warp = []
            for l, g in zip(xyzs2c, gs):
                Nl = l.shape[2]
                flow2 = _interp_post(g.reshape(3, B, Nl, -1), l)
                xyzs2_warp.append(l - flow2)
        else:
            xyzs2_warp = xyzs2c

        specs = []
        idxs = []
        for i, xw in enumerate(xyzs2_warp):
            Ni = xw.shape[2]
            idx = _knn(xw, xyz1, 16)
            idxs.append(idx)
            specs.append((_pad128(_stack_rows(xw)), _flat_idx(idx, Ni)))
        for i, xw in enumerate(xyzs2_warp):
            Ni = xw.shape[2]
            idx = idxs[i]
            boff = (jnp.arange(B, dtype=jnp.int32) * Ni)[:, None, None]
            frow = (idx + boff) * 16 + (n_idx[None, :, None] // 128)
            specs.append((cv_flat[i],
                          jnp.transpose(frow, (2, 0, 1)).reshape(-1)))
        gs = _sc_gather_multi(specs)

        costs = []
        for i in range(4):
            kxyz = gs[i].reshape(16, B, 2048, -1)[..., :3]
            rel = kxyz - jnp.transpose(xyz1, (0, 2, 1))[None]
            gflat = gs[4 + i].reshape(16, B, 16, 128, 128)
            corr = jnp.sum(gflat * lane_eye, axis=-1).reshape(16, B, 2048)
            feat = jnp.concatenate([rel, corr[..., None]], axis=-1)
            hcc = feat
            for lp in cp['cost_mlp']:
                hcc = jax.nn.relu(
                    jnp.einsum('jbnc,oc->jbno', hcc, lp['w']) + lp['b'])
            costs.append(jnp.transpose(jnp.sum(hcc, axis=0), (0, 2, 1)))
        corr = _lrelu(_conv1d(cp['merge'], jnp.concatenate(costs, axis=1)))

        lin_c1 = _conv1d(mp['conv_c1']['lin'], corr)
        lin_f1 = _conv1d(mp['conv_f1']['lin'], flow_pred)
        gs = _sc_gather_multi([
            (_pad128(_stack_rows(lin_c1)), _flat_idx(knn_idx[:, :, :16], 2048)),
            (_pad128(_stack_rows(lin_f1)), _flat_idx(knn_idx, 2048)),
        ])
        corr_feat = _dw_post(mp['conv_c1'], gs[0].reshape(16, B, 2048, -1),
                             rel32[:16])
        flow_feat = _dw_post(mp['conv_f1'], gs[1].reshape(32, B, 2048, -1),
                             rel32)
        lin_f2 = _conv1d(mp['conv_f2']['lin'], flow_feat)
        g = _sc_gather_multi([(_pad128(_stack_rows(lin_f2)),
                               _flat_idx(knn_idx[:, :, :16], 2048))])[0]
        flow_feat = _dw_post(mp['conv_f2'], g.reshape(16, B, 2048, -1),
                             rel32[:16])
        lin_mc = _conv1d(mp['conv']['lin'],
                         jnp.concatenate([corr_feat, flow_feat], axis=1))
        g = _sc_gather_multi([(_pad128(_stack_rows(lin_mc)),
                               _flat_idx(knn_idx[:, :, :16], 2048))])[0]
        mfeat = _dw_post(mp['conv'], g.reshape(16, B, 2048, -1), rel32[:16])
        motion_feat = jnp.concatenate([mfeat, flow_pred], axis=1)

        hx = jnp.concatenate([h, jnp.concatenate([x, motion_feat], axis=1)],
                             axis=1)
        lin_z = _conv1d(gp['conv_z']['lin'], hx)
        lin_r = _conv1d(gp['conv_r']['lin'], hx)
        gs = _sc_gather_multi([
            (_pad128(_stack_rows(lin_z)), _flat_idx(knn_idx[:, :, :4], 2048)),
            (_pad128(_stack_rows(lin_r)), _flat_idx(knn_idx[:, :, :4], 2048)),
        ])
        z = jax.nn.sigmoid(_dw_post(gp['conv_z'], gs[0].reshape(4, B, 2048, -1),
                                    rel32[:4], act=False))
        r = jax.nn.sigmoid(_dw_post(gp['conv_r'], gs[1].reshape(4, B, 2048, -1),
                                    rel32[:4], act=False))
        qin = jnp.concatenate([r * h, jnp.concatenate([x, motion_feat], axis=1)],
                              axis=1)
        lin_q = _conv1d(gp['conv_q']['lin'], qin)
        g = _sc_gather_multi([(_pad128(_stack_rows(lin_q)),
                               _flat_idx(knn_idx[:, :, :4], 2048))])[0]
        q = jnp.tanh(_dw_post(gp['conv_q'], g.reshape(4, B, 2048, -1),
                              rel32[:4], act=False))
        h = (1 - z) * h + z * q

        lin1 = _conv1d(fp['conv1']['lin'], h)
        g = _sc_gather_multi([(_pad128(_stack_rows(lin1)),
                               _flat_idx(knn_idx, 2048))])[0]
        f = _dw_post(fp['conv1'], g.reshape(32, B, 2048, -1), rel32)
        lin2 = _conv1d(fp['conv2']['lin'], f)
        g = _sc_gather_multi([(_pad128(_stack_rows(lin2)),
                               _flat_idx(knn_idx, 2048))])[0]
        f = _dw_post(fp['conv2'], g.reshape(32, B, 2048, -1), rel32)
        flow_delta = _conv1d(fp['fc'], f)
        flow_pred = flow_pred + flow_delta
        flow_preds.append(flow_pred)

    idx_up = _knn(xyz1, pc1, 3)
    fidx = _flat_idx(idx_up, 2048)
    specs = [(_pad128(jnp.concatenate(
        [_stack_rows(xyz1), _stack_rows(fpred)], axis=-1)), fidx)
        for fpred in flow_preds]
    gs = _sc_gather_multi(specs)
    ups = [_interp_post(g.reshape(3, B, 8192, -1), pc1) for g in gs]
    return jnp.stack(ups, axis=0)


# speedup vs baseline: 316.5707x; 6.0385x over previous
import functools
import jax
import jax.numpy as jnp
import numpy as np
from jax import lax
from jax.experimental import pallas as pl
from jax.experimental.pallas import tpu as pltpu
from jax.experimental.pallas import tpu_sc as plsc

N_ITERS = 4
_NW = 32


def _sc_gather_multi(specs):
    n = len(specs)
    budget = 98304 // n
    plans = []
    for table, idx in specs:
        V, D = table.shape
        (M,) = idx.shape
        assert D % 128 == 0 and M % (8 * _NW) == 0, (V, D, M)
        b_per_w = M // _NW
        ch = b_per_w
        while ch * D > budget or ch > 2048:
            ch //= 2
        if ch % 8 != 0 or b_per_w % ch != 0:
            ch = 8
        assert ch >= 8 and b_per_w % ch == 0, (b_per_w, ch, D)
        plans.append((V, D, M, b_per_w, ch))

    scratch = []
    for (V, D, M, b_per_w, ch) in plans:
        scratch.append(pltpu.VMEM((ch,), jnp.int32))
        scratch.append(pltpu.VMEM((ch, D), jnp.float32))
    scratch.append(pltpu.SemaphoreType.DMA)

    mesh = plsc.VectorSubcoreMesh(core_axis_name="c", subcore_axis_name="s")

    @functools.partial(
        pl.kernel, mesh=mesh,
        out_type=[jax.ShapeDtypeStruct((M, D), jnp.float32)
                  for (V, D, M, b, c) in plans],
        scratch_types=scratch,
    )
    def k(*refs):
        ins = refs[:2 * n]
        outs = refs[2 * n:3 * n]
        scr = refs[3 * n:]
        sem = scr[-1]
        wid = lax.axis_index("s") * 2 + lax.axis_index("c")
        for t in range(n):
            V, D, M, b_per_w, ch = plans[t]
            table_hbm = ins[2 * t]
            idx_hbm = ins[2 * t + 1]
            out_hbm = outs[t]
            idx_v = scr[2 * t]
            rows_v = scr[2 * t + 1]
            base = wid * b_per_w

            def body(i, _, table_hbm=table_hbm, idx_hbm=idx_hbm,
                     out_hbm=out_hbm, idx_v=idx_v, rows_v=rows_v,
                     base=base, ch=ch):
                off = base + i * ch
                pltpu.sync_copy(idx_hbm.at[pl.ds(off, ch)], idx_v)
                pltpu.async_copy(table_hbm.at[idx_v], rows_v, sem).wait()
                pltpu.sync_copy(rows_v, out_hbm.at[pl.ds(off, ch)])
                return 0

            lax.fori_loop(0, b_per_w // ch, body, 0, unroll=False)

    flat_in = []
    for table, idx in specs:
        flat_in += [table, idx]
    outs = k(*flat_in)
    return list(outs) if isinstance(outs, (list, tuple)) else [outs]


def _flat_idx(idx, V):
    B, Nq, k = idx.shape
    off = (jnp.arange(B, dtype=jnp.int32) * V)[:, None, None]
    return jnp.transpose(idx + off, (2, 0, 1)).reshape(-1)


def _stack_rows(x):
    B, C, N = x.shape
    return jnp.transpose(x, (0, 2, 1)).reshape(B * N, C)


def _pad128(t):
    D = t.shape[-1]
    Dp = ((D + 127) // 128) * 128
    if Dp == D:
        return t
    return jnp.pad(t, ((0, 0), (0, Dp - D)))


def _lrelu(x):
    return jax.nn.leaky_relu(x, 0.1)


def _knn_body(k, Ni, P, q_ref, i_ref, o_ref):
    q = q_ref[0]
    ix = i_ref[0]
    qd = jnp.dot(q, ix, preferred_element_type=jnp.float32)
    qn = q[:, 0:1] ** 2 + q[:, 1:2] ** 2 + q[:, 2:3] ** 2
    xn = ix[0:1, :] ** 2 + ix[1:2, :] ** 2 + ix[2:3, :] ** 2
    d = qn + xn - 2.0 * qd
    iota = lax.broadcasted_iota(jnp.int32, (P, Ni), 1)
    cols = []
    for _ in range(k):
        m = jnp.min(d, axis=1, keepdims=True)
        cand = jnp.where(d == m, iota, Ni)
        amin = jnp.min(cand, axis=1, keepdims=True)
        cols.append(amin)
        d = jnp.where(cand == amin, jnp.inf, d)
    o_ref[0] = jnp.concatenate(cols, axis=1)


def _knn(input_xyz, query_xyz, k, P=256):
    B, _, Ni = input_xyz.shape
    Nq = query_xyz.shape[2]
    qT = jnp.transpose(query_xyz, (0, 2, 1))
    return pl.pallas_call(
        functools.partial(_knn_body, k, Ni, P),
        grid=(B, Nq // P),
        in_specs=[
            pl.BlockSpec((1, P, 3), lambda b, t: (b, t, 0)),
            pl.BlockSpec((1, 3, Ni), lambda b, t: (b, 0, 0)),
        ],
        out_specs=pl.BlockSpec((1, P, k), lambda b, t: (b, t, 0)),
        out_shape=jax.ShapeDtypeStruct((B, Nq, k), jnp.int32),
    )(qT, input_xyz)


def _conv1d(p, x):
    return jnp.einsum('oc,bcn->bon', p['w'], x) + p['b'][None, :, None]


def _mlp1d(ps, x):
    for p in ps:
        x = _lrelu(_conv1d(p, x))
    return x


def _build_pc_pyramid(pc, n_samples):
    xyzs = [pc]
    cur = pc
    for n in n_samples:
        cur = cur[:, :, :n]
        xyzs.append(cur)
    return xyzs


def _point_conv_post(p, g, xyz_q, C):
    rel = g[..., :3] - jnp.transpose(xyz_q, (0, 2, 1))[None]
    nfeat = g[..., 3:3 + C]
    cat = jnp.concatenate([rel, nfeat], axis=-1)
    h = jnp.einsum('jbnc,oc->jbno', cat, p['w']) + p['b']
    h = h * p['gamma'] + p['beta']
    h = _lrelu(h)
    h = jnp.max(h, axis=0)
    return jnp.transpose(h, (0, 2, 1))


def _dw_post(p, nf, rel, act=True):
    C = p['wnet']['w'].shape[0]
    w = jnp.einsum('jbnd,od->jbno', rel, p['wnet']['w']) + p['wnet']['b']
    out = jnp.mean(w * nf[..., :C], axis=0)
    out = jnp.transpose(out, (0, 2, 1))
    if act:
        out = _lrelu(out)
    return out


def _interp_post(g, query_xyz):
    nxyz = g[..., :3]
    nfeat = g[..., 3:6]
    q = jnp.transpose(query_xyz, (0, 2, 1))[None]
    d = jnp.sqrt(jnp.sum((nxyz - q) ** 2, axis=-1) + 1e-12)
    w = 1.0 / (d + 1e-8)
    w = w / jnp.sum(w, axis=0, keepdims=True)
    out = jnp.sum(nfeat * w[..., None], axis=0)
    return jnp.transpose(out, (0, 2, 1))


def kernel(pc1, pc2, params):
    B = pc1.shape[0]
    xyzs1 = _build_pc_pyramid(pc1, [4096, 2048, 1024, 512, 256])
    xyzs2 = _build_pc_pyramid(pc2, [4096, 2048, 1024, 512, 256])

    def enc_level0(p, xyz0):
        return _mlp1d(p['level0'], xyz0)

    encs = [(params['fnet'], xyzs1), (params['fnet'], xyzs2),
            (params['cnet'], xyzs1)]
    f_lvl = [_mlp1d(e[0]['mlps'][0], enc_level0(e[0], e[1][0])) for e in encs]

    specs = []
    for (p, xz), f in zip(encs, f_lvl):
        idx = _knn(xz[0], xz[1], 16)
        table = _pad128(jnp.concatenate(
            [_stack_rows(xz[0]), _stack_rows(f)], axis=-1))
        specs.append((table, _flat_idx(idx, xz[0].shape[2])))
    gs = _sc_gather_multi(specs)

    feats1 = []
    for (p, xz), g in zip(encs, gs):
        Nq = xz[1].shape[2]
        gg = g.reshape(16, B, Nq, -1)
        feats1.append(_point_conv_post(p['convs'][0], gg, xz[1], 96))

    f_lvl2 = [_mlp1d(e[0]['mlps'][1], f) for e, f in zip(encs, feats1)]

    xyz1 = xyzs1[2]
    specs = []
    for (p, xz), f in zip(encs, f_lvl2):
        idx = _knn(xz[1], xz[2], 16)
        table = _pad128(jnp.concatenate(
            [_stack_rows(xz[1]), _stack_rows(f)], axis=-1))
        specs.append((table, _flat_idx(idx, xz[1].shape[2])))
    knn_idx = _knn(xyz1, xyz1, 32)
    xyz1_rows = _pad128(_stack_rows(xyz1))
    specs.append((xyz1_rows, _flat_idx(knn_idx, 2048)))
    gs = _sc_gather_multi(specs)

    feats2 = []
    for (p, xz), g in zip(encs, gs[:3]):
        gg = g.reshape(16, B, 2048, -1)
        feats2.append(_point_conv_post(p['convs'][1], gg, xz[2], 128))
    feat1, feat2, featc = feats2
    featc = _conv1d(params['cnet_aligner'], featc)

    rel32 = gs[3].reshape(32, B, 2048, -1)[..., :3] \
        - jnp.transpose(xyz1, (0, 2, 1))[None]

    cvT = [jnp.einsum('bcm,bcn->bmn', feat2, feat1) / 128.0]
    Ns2 = [2048, 1024, 512, 256]
    for i in range(1, 4):
        idx = _knn(xyzs2[2 + i - 1], xyzs2[2 + i], 3)
        table = cvT[i - 1].reshape(B * Ns2[i - 1], 2048)
        g = _sc_gather_multi([(table, _flat_idx(idx, Ns2[i - 1]))])[0]
        cvT.append(jnp.mean(g.reshape(3, B, Ns2[i], 2048), axis=0))
    cv_flat = [c.reshape(B * Ns2[i] * 16, 128) for i, c in enumerate(cvT)]

    h = jnp.tanh(featc[:, :128, :])
    x = jax.nn.relu(featc[:, 128:, :])

    xyzs2c = xyzs2[2:]
    lane_eye = jnp.eye(128, dtype=jnp.float32)
    n_idx = jnp.arange(2048, dtype=jnp.int32)

    mp = params['motion_encoder']
    gp = params['gru']
    fp = params['flow_head']
    cp = params['correlation']

    flow_preds = []
    flow_pred = jnp.zeros_like(xyz1)
    for it in range(N_ITERS):
        if it > 0:
            table = _pad128(jnp.concatenate(
                [_stack_rows(xyz1), _stack_rows(flow_pred)], axis=-1))
            specs = []
            for l in xyzs2c:
                idx = _knn(xyz1, l, 3)
                specs.append((table, _flat_idx(idx, 2048)))
            gs = _sc_gather_multi(specs)
            xyzs2_warp = []
            for l, g in zip(xyzs2c, gs):
                Nl = l.shape[2]
                flow2 = _interp_post(g.reshape(3, B, Nl, -1), l)
                xyzs2_warp.append(l - flow2)
        else:
            xyzs2_warp = xyzs2c

        specs = []
        idxs = []
        for i, xw in enumerate(xyzs2_warp):
            Ni = xw.shape[2]
            idx = _knn(xw, xyz1, 16)
            idxs.append(idx)
            specs.append((_pad128(_stack_rows(xw)), _flat_idx(idx, Ni)))
        for i, xw in enumerate(xyzs2_warp):
            Ni = xw.shape[2]
            idx = idxs[i]
            boff = (jnp.arange(B, dtype=jnp.int32) * Ni)[:, None, None]
            frow = (idx + boff) * 16 + (n_idx[None, :, None] // 128)
            specs.append((cv_flat[i],
                          jnp.transpose(frow, (2, 0, 1)).reshape(-1)))
        gs = _sc_gather_multi(specs)

        costs = []
        for i in range(4):
            kxyz = gs[i].reshape(16, B, 2048, -1)[..., :3]
            rel = kxyz - jnp.transpose(xyz1, (0, 2, 1))[None]
            gflat = gs[4 + i].reshape(16, B, 16, 128, 128)
            corr = jnp.sum(gflat * lane_eye, axis=-1).reshape(16, B, 2048)
            feat = jnp.concatenate([rel, corr[..., None]], axis=-1)
            hcc = feat
            for lp in cp['cost_mlp']:
                hcc = jax.nn.relu(
                    jnp.einsum('jbnc,oc->jbno', hcc, lp['w']) + lp['b'])
            costs.append(jnp.transpose(jnp.sum(hcc, axis=0), (0, 2, 1)))
        corr = _lrelu(_conv1d(cp['merge'], jnp.concatenate(costs, axis=1)))

        lin_c1 = _conv1d(mp['conv_c1']['lin'], corr)
        lin_f1 = _conv1d(mp['conv_f1']['lin'], flow_pred)
        gs = _sc_gather_multi([
            (_pad128(_stack_rows(lin_c1)), _flat_idx(knn_idx[:, :, :16], 2048)),
            (_pad128(_stack_rows(lin_f1)), _flat_idx(knn_idx, 2048)),
        ])
        corr_feat = _dw_post(mp['conv_c1'], gs[0].reshape(16, B, 2048, -1),
                             rel32[:16])
        flow_feat = _dw_post(mp['conv_f1'], gs[1].reshape(32, B, 2048, -1),
                             rel32)
        lin_f2 = _conv1d(mp['conv_f2']['lin'], flow_feat)
        g = _sc_gather_multi([(_pad128(_stack_rows(lin_f2)),
                               _flat_idx(knn_idx[:, :, :16], 2048))])[0]
        flow_feat = _dw_post(mp['conv_f2'], g.reshape(16, B, 2048, -1),
                             rel32[:16])
        lin_mc = _conv1d(mp['conv']['lin'],
                         jnp.concatenate([corr_feat, flow_feat], axis=1))
        g = _sc_gather_multi([(_pad128(_stack_rows(lin_mc)),
                               _flat_idx(knn_idx[:, :, :16], 2048))])[0]
        mfeat = _dw_post(mp['conv'], g.reshape(16, B, 2048, -1), rel32[:16])
        motion_feat = jnp.concatenate([mfeat, flow_pred], axis=1)

        hx = jnp.concatenate([h, jnp.concatenate([x, motion_feat], axis=1)],
                             axis=1)
        lin_z = _conv1d(gp['conv_z']['lin'], hx)
        lin_r = _conv1d(gp['conv_r']['lin'], hx)
        gs = _sc_gather_multi([
            (_pad128(_stack_rows(lin_z)), _flat_idx(knn_idx[:, :, :4], 2048)),
            (_pad128(_stack_rows(lin_r)), _flat_idx(knn_idx[:, :, :4], 2048)),
        ])
        z = jax.nn.sigmoid(_dw_post(gp['conv_z'], gs[0].reshape(4, B, 2048, -1),
                                    rel32[:4], act=False))
        r = jax.nn.sigmoid(_dw_post(gp['conv_r'], gs[1].reshape(4, B, 2048, -1),
                                    rel32[:4], act=False))
        qin = jnp.concatenate([r * h, jnp.concatenate([x, motion_feat], axis=1)],
                              axis=1)
        lin_q = _conv1d(gp['conv_q']['lin'], qin)
        g = _sc_gather_multi([(_pad128(_stack_rows(lin_q)),
                               _flat_idx(knn_idx[:, :, :4], 2048))])[0]
        q = jnp.tanh(_dw_post(gp['conv_q'], g.reshape(4, B, 2048, -1),
                              rel32[:4], act=False))
        h = (1 - z) * h + z * q

        lin1 = _conv1d(fp['conv1']['lin'], h)
        g = _sc_gather_multi([(_pad128(_stack_rows(lin1)),
                               _flat_idx(knn_idx, 2048))])[0]
        f = _dw_post(fp['conv1'], g.reshape(32, B, 2048, -1), rel32)
        lin2 = _conv1d(fp['conv2']['lin'], f)
        g = _sc_gather_multi([(_pad128(_stack_rows(lin2)),
                               _flat_idx(knn_idx, 2048))])[0]
        f = _dw_post(fp['conv2'], g.reshape(32, B, 2048, -1), rel32)
        flow_delta = _conv1d(fp['fc'], f)
        flow_pred = flow_pred + flow_delta
        flow_preds.append(flow_pred)

    idx_up = _knn(xyz1, pc1, 3)
    fidx = _flat_idx(idx_up, 2048)
    specs = [(_pad128(jnp.concatenate(
        [_stack_rows(xyz1), _stack_rows(fpred)], axis=-1)), fidx)
        for fpred in flow_preds]
    gs = _sc_gather_multi(specs)
    ups = [_interp_post(g.reshape(3, B, 8192, -1), pc1) for g in gs]
    return jnp.stack(ups, axis=0)
